# Initial kernel scaffold; baseline (speedup 1.0000x reference)
import jax
import jax.numpy as jnp
from jax.experimental import pallas as pl


def kernel(x, edge_index, batch, W_in, b_in, W, b, gamma, beta):
    raise NotImplementedError("write your pallas kernel here")


# speedup vs baseline: 6.5769x; 6.5769x over previous
import functools

import jax
import jax.numpy as jnp
from jax import lax
from jax.experimental import pallas as pl
from jax.experimental.pallas import tpu as pltpu
from jax.experimental.pallas import tpu_sc as plsc

_NC = 2
_NS = 16
_NW = _NC * _NS

_ROWS_PER_BLOCK = 400
_NUM_GRAPHS = 64


def _make_edge_aggregate(N, H, E, K):
    epw = E // _NW
    nchunk = epw // K
    rpt = -(-N // (_NS * 8)) * 8
    NP = rpt * _NS

    @functools.partial(
        pl.kernel,
        mesh=plsc.VectorSubcoreMesh(core_axis_name="c", subcore_axis_name="s"),
        out_type=jax.ShapeDtypeStruct((_NC * NP, H), jnp.float32),
        scratch_types=[
            pltpu.VMEM((nchunk, K), jnp.int32),
            pltpu.VMEM((nchunk, K), jnp.int32),
            pltpu.VMEM((K, H), jnp.float32),
            pltpu.VMEM_SHARED((NP, H), jnp.float32),
            pltpu.SemaphoreType.DMA,
        ],
    )
    def edge_aggregate(hr_hbm, src_hbm, dst_hbm, zeros_hbm, out_hbm,
                       src_v, dst_v, rows_v, acc_sh, sem):
        c = lax.axis_index("c")
        s = lax.axis_index("s")
        wid = s * _NC + c
        pltpu.sync_copy(zeros_hbm, acc_sh.at[pl.ds(s * rpt, rpt)])
        pltpu.sync_copy(src_hbm.at[wid], src_v)
        pltpu.sync_copy(dst_hbm.at[wid], dst_v)
        plsc.subcore_barrier()

        def body(j, carry):
            pltpu.async_copy(hr_hbm.at[src_v.at[j]], rows_v, sem).wait()
            pltpu.sync_copy(rows_v, acc_sh.at[dst_v.at[j]], add=True)
            return carry

        lax.fori_loop(0, nchunk, body, 0)
        plsc.subcore_barrier()
        pltpu.sync_copy(acc_sh.at[pl.ds(s * rpt, rpt)],
                        out_hbm.at[pl.ds(c * NP + s * rpt, rpt)])

    return edge_aggregate, NP


def _proj_body(x_ref, w_ref, b_ref, h_ref, hr_ref):
    z = jnp.dot(x_ref[...], w_ref[...], preferred_element_type=jnp.float32)
    z = z + b_ref[...]
    h = z * (1.0 / (1.0 + jnp.exp(-z)))
    h_ref[...] = h
    hr_ref[...] = jnp.maximum(h, 0.0)


def _layer_body(h_ref, p_ref, w_ref, b_ref, g_ref, be_ref, hn_ref, hrn_ref):
    h = h_ref[...]
    a = h + p_ref[0] + p_ref[1]
    z = jnp.dot(a, w_ref[...], preferred_element_type=jnp.float32) + b_ref[...]
    mu = jnp.mean(z, axis=-1, keepdims=True)
    zc = z - mu
    var = jnp.mean(zc * zc, axis=-1, keepdims=True)
    zn = zc * lax.rsqrt(var + 1e-5) * g_ref[...] + be_ref[...]
    act = zn * (1.0 / (1.0 + jnp.exp(-zn)))
    out = act + h
    hn_ref[...] = out
    hrn_ref[...] = jnp.maximum(out, 0.0)


def _pool_body(h_ref, b_ref, out_ref, sums, counts):
    i = pl.program_id(0)

    @pl.when(i == 0)
    def _():
        sums[...] = jnp.zeros_like(sums)
        counts[...] = jnp.zeros_like(counts)

    bvec = b_ref[0]
    gids = lax.broadcasted_iota(jnp.int32, (_NUM_GRAPHS, bvec.shape[1]), 0)
    onehot = (bvec == gids).astype(jnp.float32)
    sums[...] += lax.dot_general(onehot, h_ref[...], (((1,), (0,)), ((), ())),
                                 preferred_element_type=jnp.float32)
    counts[...] = counts[...] + jnp.sum(onehot, axis=1, keepdims=True)

    @pl.when(i == pl.num_programs(0) - 1)
    def _():
        out_ref[...] = sums[...] / jnp.maximum(counts[...], 1.0)


def kernel(x, edge_index, batch, W_in, b_in, W, b, gamma, beta):
    N, D = x.shape
    H = W_in.shape[1]
    L = W.shape[0]
    E = edge_index.shape[1]
    RB = _ROWS_PER_BLOCK
    nb = N // RB

    K = 80
    epw = E // _NW
    nchunk = epw // K
    src3 = edge_index[0].reshape(_NW, nchunk, K)
    dst3 = edge_index[1].reshape(_NW, nchunk, K)

    edge_aggregate, NP = _make_edge_aggregate(N, H, E, K)
    rpt = NP // _NS
    zeros = jnp.zeros((rpt, H), jnp.float32)

    full = lambda i: (0, 0)
    row_spec = pl.BlockSpec((RB, H), lambda i: (i, 0))

    proj = pl.pallas_call(
        _proj_body,
        grid=(nb,),
        in_specs=[row_spec,
                  pl.BlockSpec((D, H), full),
                  pl.BlockSpec((1, H), full)],
        out_specs=[row_spec, row_spec],
        out_shape=[jax.ShapeDtypeStruct((N, H), jnp.float32)] * 2,
    )

    layer = pl.pallas_call(
        _layer_body,
        grid=(nb,),
        in_specs=[row_spec,
                  pl.BlockSpec((_NC, RB, H), lambda i: (0, i, 0)),
                  pl.BlockSpec((H, H), full),
                  pl.BlockSpec((1, H), full),
                  pl.BlockSpec((1, H), full),
                  pl.BlockSpec((1, H), full)],
        out_specs=[row_spec, row_spec],
        out_shape=[jax.ShapeDtypeStruct((N, H), jnp.float32)] * 2,
    )

    pool = pl.pallas_call(
        _pool_body,
        grid=(nb,),
        in_specs=[row_spec,
                  pl.BlockSpec((1, 1, RB), lambda i: (i, 0, 0))],
        out_specs=pl.BlockSpec((_NUM_GRAPHS, H), full),
        out_shape=jax.ShapeDtypeStruct((_NUM_GRAPHS, H), jnp.float32),
        scratch_shapes=[pltpu.VMEM((_NUM_GRAPHS, H), jnp.float32),
                        pltpu.VMEM((_NUM_GRAPHS, H), jnp.float32)],
    )

    h, hr = proj(x, W_in, b_in.reshape(1, H))
    for l in range(L):
        part = edge_aggregate(hr, src3, dst3, zeros)
        part = part.reshape(_NC, NP, H)[:, :N, :]
        h, hr = layer(h, part, W[l], b[l].reshape(1, H),
                      gamma[l].reshape(1, H), beta[l].reshape(1, H))
    batch3 = batch.reshape(nb, 1, RB)
    return pool(h, batch3)
